# baseline (speedup 1.0000x reference)
import jax
import jax.numpy as jnp
from jax.experimental import pallas as pl

_E = 16
_BLK = 512


def _moe_body(x_ref, w_ref, b_ref, o_ref):
    xb = x_ref[...]
    wt = w_ref[...]
    acc = jax.lax.dot_general(wt, xb, (((1,), (1,)), ((), ())),
                              preferred_element_type=jnp.float32)
    acc = acc + b_ref[...]
    nz = acc[:_E, :]
    eo = acc[_E:, :]
    iota = jax.lax.broadcasted_iota(jnp.int32, nz.shape, 0).astype(jnp.float32)
    v1 = jnp.max(nz, axis=0, keepdims=True)
    i1 = jnp.min(jnp.where(nz == v1, iota, float(_E)), axis=0, keepdims=True)
    m1 = iota == i1
    masked = jnp.where(m1, -jnp.inf, nz)
    v2 = jnp.max(masked, axis=0, keepdims=True)
    i2 = jnp.min(jnp.where(masked == v2, iota, float(_E)), axis=0, keepdims=True)
    m2 = iota == i2
    t = jnp.exp(v2 - v1)
    rcp = 1.0 / (1.0 + t)
    w1 = rcp
    w2 = t * rcp
    sig = jax.nn.sigmoid(eo)
    coef = jnp.where(m1, w1, jnp.where(m2, w2, 0.0))
    o_ref[...] = jnp.sum(coef * sig, axis=0, keepdims=True)


def kernel(x, W_route, b_route, W_noise, b_noise, W_experts, b_experts):
    n, d = x.shape
    wt = jnp.concatenate([W_noise.T, W_experts], axis=0)
    bt = jnp.concatenate([b_noise, b_experts])[:, None]
    out = pl.pallas_call(
        _moe_body,
        grid=(n // _BLK,),
        in_specs=[
            pl.BlockSpec((_BLK, d), lambda i: (i, 0)),
            pl.BlockSpec((2 * _E, d), lambda i: (0, 0)),
            pl.BlockSpec((2 * _E, 1), lambda i: (0, 0)),
        ],
        out_specs=pl.BlockSpec((1, _BLK), lambda i: (0, i)),
        out_shape=jax.ShapeDtypeStruct((1, n), jnp.float32),
    )(x, wt, bt)
    return out.reshape(n, 1)


# speedup vs baseline: 1.1369x; 1.1369x over previous
import jax
import jax.numpy as jnp
from jax.experimental import pallas as pl

_E = 16
_HB = 512


def _moe_body(xa_ref, xb_ref, w_ref, b_ref, o_ref):
    wt = w_ref[...]
    acca = jax.lax.dot_general(wt, xa_ref[...], (((1,), (1,)), ((), ())),
                               preferred_element_type=jnp.float32)
    accb = jax.lax.dot_general(wt, xb_ref[...], (((1,), (1,)), ((), ())),
                               preferred_element_type=jnp.float32)
    acc = jnp.concatenate([acca, accb], axis=1) + b_ref[...]
    nz = acc[:_E, :]
    eo = acc[_E:, :]
    iota = jax.lax.broadcasted_iota(jnp.int32, nz.shape, 0).astype(jnp.float32)
    v1 = jnp.max(nz, axis=0, keepdims=True)
    i1 = jnp.min(jnp.where(nz == v1, iota, float(_E)), axis=0, keepdims=True)
    m1 = iota == i1
    masked = jnp.where(m1, -jnp.inf, nz)
    v2 = jnp.max(masked, axis=0, keepdims=True)
    i2 = jnp.min(jnp.where(masked == v2, iota, float(_E)), axis=0, keepdims=True)
    m2 = iota == i2
    t = jnp.exp(v2 - v1)
    rcp = 1.0 / (1.0 + t)
    sig = jax.nn.sigmoid(eo)
    coef = jnp.where(m1, rcp, jnp.where(m2, t * rcp, 0.0))
    o_ref[...] = jnp.sum(coef * sig, axis=0, keepdims=True)


def kernel(x, W_route, b_route, W_noise, b_noise, W_experts, b_experts):
    n, d = x.shape
    wt = jnp.concatenate([W_noise.T, W_experts], axis=0)
    bt = jnp.concatenate([b_noise, b_experts])[:, None]
    grid = n // (2 * _HB)
    out = pl.pallas_call(
        _moe_body,
        grid=(grid,),
        in_specs=[
            pl.BlockSpec((_HB, d), lambda i: (2 * i, 0)),
            pl.BlockSpec((_HB, d), lambda i: (2 * i + 1, 0)),
            pl.BlockSpec((2 * _E, d), lambda i: (0, 0)),
            pl.BlockSpec((2 * _E, 1), lambda i: (0, 0)),
        ],
        out_specs=pl.BlockSpec((1, 2 * _HB), lambda i: (0, i)),
        out_shape=jax.ShapeDtypeStruct((1, n), jnp.float32),
    )(x, x, wt, bt)
    return out.reshape(n, 1)


# speedup vs baseline: 1.1378x; 1.0008x over previous
import jax
import jax.numpy as jnp
from jax.experimental import pallas as pl

_E = 16
_BLK = 1024


def _moe_body(x_ref, w_ref, b_ref, o_ref):
    xb = x_ref[...]
    wt = w_ref[...]
    acc = jax.lax.dot_general(wt, xb, (((1,), (1,)), ((), ())),
                              preferred_element_type=jnp.float32)
    acc = acc + b_ref[...]
    nz = acc[:_E, :]
    eo = acc[_E:, :]
    iota = jax.lax.broadcasted_iota(jnp.int32, nz.shape, 0).astype(jnp.float32)
    v1 = jnp.max(nz, axis=0, keepdims=True)
    i1 = jnp.min(jnp.where(nz == v1, iota, float(_E)), axis=0, keepdims=True)
    m1 = iota == i1
    masked = jnp.where(m1, -jnp.inf, nz)
    v2 = jnp.max(masked, axis=0, keepdims=True)
    i2 = jnp.min(jnp.where(masked == v2, iota, float(_E)), axis=0, keepdims=True)
    m2 = iota == i2
    t = jnp.exp(v2 - v1)
    rcp = 1.0 / (1.0 + t)
    w1 = rcp
    w2 = t * rcp
    sig = jax.nn.sigmoid(eo)
    coef = jnp.where(m1, w1, jnp.where(m2, w2, 0.0))
    o_ref[...] = jnp.sum(coef * sig, axis=0, keepdims=True)


def kernel(x, W_route, b_route, W_noise, b_noise, W_experts, b_experts):
    n, d = x.shape
    wt = jnp.concatenate([W_noise.T, W_experts], axis=0)
    bt = jnp.concatenate([b_noise, b_experts])[:, None]
    out = pl.pallas_call(
        _moe_body,
        grid=(n // _BLK,),
        in_specs=[
            pl.BlockSpec((_BLK, d), lambda i: (i, 0)),
            pl.BlockSpec((2 * _E, d), lambda i: (0, 0)),
            pl.BlockSpec((2 * _E, 1), lambda i: (0, 0)),
        ],
        out_specs=pl.BlockSpec((1, _BLK), lambda i: (0, i)),
        out_shape=jax.ShapeDtypeStruct((1, n), jnp.float32),
    )(x, wt, bt)
    return out.reshape(n, 1)
